# Initial kernel scaffold; baseline (speedup 1.0000x reference)
import jax
import jax.numpy as jnp
from jax.experimental import pallas as pl


def kernel(boxes, scores):
    raise NotImplementedError("write your pallas kernel here")


# speedup vs baseline: 18.3574x; 18.3574x over previous
import jax
import jax.numpy as jnp
from jax.experimental import pallas as pl
from jax.experimental.pallas import tpu as pltpu

_N = 20000
_CONF = 0.35
_IOU = 0.5
_MAXDET = 1000
_BR, _BC = 8, 128
_BSZ = _BR * _BC
_NB = (_N + _BSZ - 1) // _BSZ
_PAD = _NB * _BSZ


def _nms_body(x1_ref, y1_ref, x2_ref, y2_ref, sc_ref,
              ocx_ref, ocy_ref, ow_ref, oh_ref, osc_ref, ov_ref,
              live_ref):
    iota = (jax.lax.broadcasted_iota(jnp.int32, (_BR, _BC), 0) * _BC
            + jax.lax.broadcasted_iota(jnp.int32, (_BR, _BC), 1))
    neg = jnp.float32(-1.0)
    big = jnp.int32(1 << 30)

    bm = jnp.full((_BR, _BC), neg, jnp.float32)
    for b in range(_NB):
        sb = sc_ref[b]
        lv = jnp.where(sb > _CONF, sb, neg)
        live_ref[b] = lv
        bm = jnp.where(iota == b, jnp.max(lv), bm)

    zf = jnp.zeros((_BR, _BC), jnp.float32)

    def cond(c):
        return (c[1] > 0.0) & (c[0] < _MAXDET)

    def body(c):
        (k, m, bm, kx1, ky1, kx2, ky2, ka,
         ocx, ocy, ow, oh, osc, ov) = c
        bidx = jnp.min(jnp.where(bm == m, iota, big))
        sb = live_ref[bidx]
        fidx = jnp.min(jnp.where(sb == m, iota, big))
        sel = iota == fidx
        bx1 = jnp.sum(jnp.where(sel, x1_ref[bidx], 0.0))
        by1 = jnp.sum(jnp.where(sel, y1_ref[bidx], 0.0))
        bx2 = jnp.sum(jnp.where(sel, x2_ref[bidx], 0.0))
        by2 = jnp.sum(jnp.where(sel, y2_ref[bidx], 0.0))
        ix1 = jnp.maximum(bx1, kx1)
        iy1 = jnp.maximum(by1, ky1)
        ix2 = jnp.minimum(bx2, kx2)
        iy2 = jnp.minimum(by2, ky2)
        inter = jnp.maximum(ix2 - ix1, 0.0) * jnp.maximum(iy2 - iy1, 0.0)
        w = bx2 - bx1
        h = by2 - by1
        a1 = w * h
        iou = inter / (a1 + ka - inter + 1e-9)
        keep = jnp.max(jnp.where(iou > _IOU, 1.0, 0.0)) <= 0.0
        slot = jnp.logical_and(iota == k, keep)
        kx1 = jnp.where(slot, bx1, kx1)
        ky1 = jnp.where(slot, by1, ky1)
        kx2 = jnp.where(slot, bx2, kx2)
        ky2 = jnp.where(slot, by2, ky2)
        ka = jnp.where(slot, a1, ka)
        ocx = jnp.where(slot, bx1 + w / 2.0, ocx)
        ocy = jnp.where(slot, by1 + h / 2.0, ocy)
        ow = jnp.where(slot, w, ow)
        oh = jnp.where(slot, h, oh)
        osc = jnp.where(slot, m, osc)
        ov = jnp.where(slot, 1.0, ov)
        k = k + keep.astype(jnp.int32)
        sb = jnp.where(sel, neg, sb)
        live_ref[bidx] = sb
        bm = jnp.where(iota == bidx, jnp.max(sb), bm)
        m = jnp.max(bm)
        return (k, m, bm, kx1, ky1, kx2, ky2, ka,
                ocx, ocy, ow, oh, osc, ov)

    init = (jnp.int32(0), jnp.max(bm), bm, zf, zf, zf, zf, zf,
            zf, zf, zf, zf, zf, zf)
    res = jax.lax.while_loop(cond, body, init)
    ocx_ref[...] = res[8]
    ocy_ref[...] = res[9]
    ow_ref[...] = res[10]
    oh_ref[...] = res[11]
    osc_ref[...] = res[12]
    ov_ref[...] = res[13]


def kernel(boxes, scores):
    pad = _PAD - _N
    shp = (_NB, _BR, _BC)
    x1 = jnp.pad(boxes[:, 0], (0, pad)).reshape(shp)
    y1 = jnp.pad(boxes[:, 1], (0, pad)).reshape(shp)
    x2 = jnp.pad(boxes[:, 2], (0, pad)).reshape(shp)
    y2 = jnp.pad(boxes[:, 3], (0, pad)).reshape(shp)
    sc = jnp.pad(scores, (0, pad)).reshape(shp)
    outs = pl.pallas_call(
        _nms_body,
        out_shape=[jax.ShapeDtypeStruct((_BR, _BC), jnp.float32)] * 6,
        scratch_shapes=[pltpu.VMEM(shp, jnp.float32)],
    )(x1, y1, x2, y2, sc)
    cols = [o.reshape(-1)[:_MAXDET] for o in outs]
    return jnp.stack(cols, axis=-1)


# speedup vs baseline: 23.4154x; 1.2755x over previous
import jax
import jax.numpy as jnp
from jax.experimental import pallas as pl
from jax.experimental.pallas import tpu as pltpu

_N = 20000
_CONF = 0.35
_IOU = 0.5
_MAXDET = 1000
_NR, _NC = 256, 128
_BR, _BC = 8, 128
_BSZ = _BR * _BC
_NPAD = _NR * _NC


def _nms_body(x1_ref, y1_ref, x2_ref, y2_ref, sc_ref,
              ocx_ref, ocy_ref, ow_ref, oh_ref, osc_ref, ov_ref,
              k_ref, sx1_ref, sy1_ref, sx2_ref, sy2_ref):
    f = (jax.lax.broadcasted_iota(jnp.int32, (_NR, _NC), 0) * _NC
         + jax.lax.broadcasted_iota(jnp.int32, (_NR, _NC), 1))

    sc = sc_ref[...]
    key = jnp.where(sc > _CONF, sc, -1.0)
    idx = f
    x1 = x1_ref[...]
    y1 = y1_ref[...]
    x2 = x2_ref[...]
    y2 = y2_ref[...]

    def _exchange(s, kk, j, fetch):
        key, idx, x1, y1, x2, y2 = s
        lob = (f & j) == 0
        pk = fetch(key, lob)
        pi = fetch(idx, lob)
        pless = (pk > key) | ((pk == key) & (pi < idx))
        dirdesc = (f & kk) != 0
        take = jnp.logical_xor(jnp.logical_xor(pless, lob),
                               jnp.logical_not(dirdesc))
        return (jnp.where(take, pk, key),
                jnp.where(take, pi, idx),
                jnp.where(take, fetch(x1, lob), x1),
                jnp.where(take, fetch(y1, lob), y1),
                jnp.where(take, fetch(x2, lob), x2),
                jnp.where(take, fetch(y2, lob), y2))

    def _sub_body(c):
        kk, j = c[0], c[1]

        def fetch(x, lob):
            d = jax.lax.shift_right_logical(j, 7)
            return jnp.where(lob, pltpu.roll(x, _NR - d, axis=0),
                             pltpu.roll(x, d, axis=0))

        return (kk, jax.lax.shift_right_logical(j, 1),
                *_exchange(c[2:], kk, j, fetch))

    def _lane_body(c):
        kk, j = c[0], c[1]

        def fetch(x, lob):
            return jnp.where(lob, pltpu.roll(x, _NC - j, axis=1),
                             pltpu.roll(x, j, axis=1))

        return (kk, jax.lax.shift_right_logical(j, 1),
                *_exchange(c[2:], kk, j, fetch))

    def _level_body(c):
        kk = c[0]
        j0 = jax.lax.shift_right_logical(kk, 1)
        c = jax.lax.while_loop(lambda t: t[1] >= _NC, _sub_body,
                               (kk, j0) + c[1:])
        c = jax.lax.while_loop(lambda t: t[1] >= 1, _lane_body, c)
        return (jax.lax.shift_left(kk, 1),) + c[2:]

    res = jax.lax.while_loop(lambda t: t[0] <= _NPAD, _level_body,
                             (jnp.int32(2), key, idx, x1, y1, x2, y2))
    key, _, x1, y1, x2, y2 = res[1:]

    k_ref[...] = key
    sx1_ref[...] = x1
    sy1_ref[...] = y1
    sx2_ref[...] = x2
    sy2_ref[...] = y2

    g = (jax.lax.broadcasted_iota(jnp.int32, (_BR, _BC), 0) * _BC
         + jax.lax.broadcasted_iota(jnp.int32, (_BR, _BC), 1))
    zf = jnp.zeros((_BR, _BC), jnp.float32)

    sel0 = f == 0
    s0 = jnp.sum(jnp.where(sel0, key, 0.0))
    bx10 = jnp.sum(jnp.where(sel0, x1, 0.0))
    by10 = jnp.sum(jnp.where(sel0, y1, 0.0))
    bx20 = jnp.sum(jnp.where(sel0, x2, 0.0))
    by20 = jnp.sum(jnp.where(sel0, y2, 0.0))

    def cond(c):
        return (c[2] > 0.0) & (c[1] < _MAXDET)

    def body(c):
        (p, k, s, bx1, by1, bx2, by2, kx1, ky1, kx2, ky2, ka,
         ocx, ocy, ow, oh, osc, ov) = c
        ix1 = jnp.maximum(bx1, kx1)
        iy1 = jnp.maximum(by1, ky1)
        ix2 = jnp.minimum(bx2, kx2)
        iy2 = jnp.minimum(by2, ky2)
        inter = jnp.maximum(ix2 - ix1, 0.0) * jnp.maximum(iy2 - iy1, 0.0)
        w = bx2 - bx1
        h = by2 - by1
        a1 = w * h
        iou = inter / (a1 + ka - inter + 1e-9)
        gt = jnp.where(iou > _IOU, 1.0, 0.0)
        t = gt
        for sh in (64, 32, 16, 8, 4, 2, 1):
            t = jnp.maximum(t, pltpu.roll(t, sh, axis=1))
        for sh in (4, 2, 1):
            t = jnp.maximum(t, pltpu.roll(t, sh, axis=0))
        keepv = t < 0.5
        slot = jnp.logical_and(g == k, keepv)
        kx1 = jnp.where(slot, bx1, kx1)
        ky1 = jnp.where(slot, by1, ky1)
        kx2 = jnp.where(slot, bx2, kx2)
        ky2 = jnp.where(slot, by2, ky2)
        ka = jnp.where(slot, a1, ka)
        ocx = jnp.where(slot, bx1 + w / 2.0, ocx)
        ocy = jnp.where(slot, by1 + h / 2.0, ocy)
        ow = jnp.where(slot, w, ow)
        oh = jnp.where(slot, h, oh)
        osc = jnp.where(slot, s, osc)
        ov = jnp.where(slot, 1.0, ov)
        keep_s = jnp.max(gt) < 0.5
        k = k + keep_s.astype(jnp.int32)
        pn = p + 1
        rs = jax.lax.shift_left(jax.lax.shift_right_logical(pn, 10), 3)
        sel = g == (pn & (_BSZ - 1))
        sn = jnp.sum(jnp.where(sel, k_ref[pl.ds(rs, _BR), :], 0.0))
        nx1 = jnp.sum(jnp.where(sel, sx1_ref[pl.ds(rs, _BR), :], 0.0))
        ny1 = jnp.sum(jnp.where(sel, sy1_ref[pl.ds(rs, _BR), :], 0.0))
        nx2 = jnp.sum(jnp.where(sel, sx2_ref[pl.ds(rs, _BR), :], 0.0))
        ny2 = jnp.sum(jnp.where(sel, sy2_ref[pl.ds(rs, _BR), :], 0.0))
        return (pn, k, sn, nx1, ny1, nx2, ny2, kx1, ky1, kx2, ky2, ka,
                ocx, ocy, ow, oh, osc, ov)

    init = (jnp.int32(0), jnp.int32(0), s0, bx10, by10, bx20, by20,
            zf, zf, zf, zf, zf, zf, zf, zf, zf, zf, zf)
    res = jax.lax.while_loop(cond, body, init)
    ocx_ref[...] = res[12]
    ocy_ref[...] = res[13]
    ow_ref[...] = res[14]
    oh_ref[...] = res[15]
    osc_ref[...] = res[16]
    ov_ref[...] = res[17]


def kernel(boxes, scores):
    pad = _NPAD - _N
    shp = (_NR, _NC)
    x1 = jnp.pad(boxes[:, 0], (0, pad)).reshape(shp)
    y1 = jnp.pad(boxes[:, 1], (0, pad)).reshape(shp)
    x2 = jnp.pad(boxes[:, 2], (0, pad)).reshape(shp)
    y2 = jnp.pad(boxes[:, 3], (0, pad)).reshape(shp)
    sc = jnp.pad(scores, (0, pad)).reshape(shp)
    outs = pl.pallas_call(
        _nms_body,
        out_shape=[jax.ShapeDtypeStruct((_BR, _BC), jnp.float32)] * 6,
        scratch_shapes=[pltpu.VMEM(shp, jnp.float32)] * 5,
    )(x1, y1, x2, y2, sc)
    cols = [o.reshape(-1)[:_MAXDET] for o in outs]
    return jnp.stack(cols, axis=-1)
